# Initial kernel scaffold; baseline (speedup 1.0000x reference)
import jax
import jax.numpy as jnp
from jax.experimental import pallas as pl


def kernel(h_i, R_i, t_i, v_i, entity_emb, relation_emb):
    raise NotImplementedError("write your pallas kernel here")


# speedup vs baseline: 1.0855x; 1.0855x over previous
import functools

import jax
import jax.numpy as jnp
from jax import lax
from jax.experimental import pallas as pl
from jax.experimental.pallas import tpu as pltpu
from jax.experimental.pallas import tpu_sc as plsc

NC = 2
NS = 16
NW = NC * NS

B = 65536
BV = 1024
DIM = 16
RDIM = 256

B_W = B // NW
BV_W = BV // NW
CHUNK = 128
NCH = B_W // CHUNK
R_CHUNK = 256
R_STEPS = B_W // R_CHUNK


def _sc_body(ent_hbm, rel_hbm, h_hbm, r_hbm, t_hbm, v_hbm,
             hs_out, rs_out, ts_out, vs_out,
             idx_v, rows_v, vidx_v, vrows_v, rrows_v, sem):
    wid = lax.axis_index("s") * NC + lax.axis_index("c")
    base = wid * B_W

    for src_hbm, dst_hbm in ((h_hbm, hs_out), (t_hbm, ts_out)):
        pltpu.sync_copy(src_hbm.at[pl.ds(wid * NCH, NCH)], idx_v)
        copies = []
        for j in range(NCH):
            copies.append(pltpu.async_copy(
                ent_hbm.at[idx_v.at[j]],
                rows_v.at[pl.ds(j * CHUNK, CHUNK)], sem))
        for c in copies:
            c.wait()
        pltpu.sync_copy(rows_v, dst_hbm.at[pl.ds(base, B_W)])

    pltpu.sync_copy(v_hbm.at[pl.ds(wid * BV_W, BV_W)], vidx_v)
    pltpu.async_copy(ent_hbm.at[vidx_v], vrows_v, sem).wait()
    pltpu.sync_copy(vrows_v, vs_out.at[pl.ds(wid * BV_W, BV_W)])

    pltpu.sync_copy(r_hbm.at[pl.ds(wid * NCH, NCH)], idx_v)
    for c in range(R_STEPS):
        g0 = pltpu.async_copy(rel_hbm.at[idx_v.at[2 * c]],
                              rrows_v.at[pl.ds(0, CHUNK)], sem)
        g1 = pltpu.async_copy(rel_hbm.at[idx_v.at[2 * c + 1]],
                              rrows_v.at[pl.ds(CHUNK, CHUNK)], sem)
        g0.wait()
        g1.wait()
        pltpu.sync_copy(rrows_v, rs_out.at[pl.ds(base + c * R_CHUNK, R_CHUNK)])


@jax.jit
def _run(h_flat, r_flat, t_flat, v_i, entity_emb, rel_flat):
    mesh = plsc.VectorSubcoreMesh(core_axis_name="c", subcore_axis_name="s")
    f = functools.partial(
        pl.kernel,
        mesh=mesh,
        out_type=[
            jax.ShapeDtypeStruct((B, DIM), jnp.float32),
            jax.ShapeDtypeStruct((B, RDIM), jnp.float32),
            jax.ShapeDtypeStruct((B, DIM), jnp.float32),
            jax.ShapeDtypeStruct((BV, DIM), jnp.float32),
        ],
        scratch_types=[
            pltpu.VMEM((NCH, CHUNK), jnp.int32),
            pltpu.VMEM((B_W, DIM), jnp.float32),
            pltpu.VMEM((BV_W,), jnp.int32),
            pltpu.VMEM((BV_W, DIM), jnp.float32),
            pltpu.VMEM((R_CHUNK, RDIM), jnp.float32),
            pltpu.SemaphoreType.DMA,
        ],
        compiler_params=pltpu.CompilerParams(use_tc_tiling_on_sc=False),
    )(_sc_body)
    return f(entity_emb, rel_flat, h_flat, r_flat, t_flat, v_i)


def kernel(h_i, R_i, t_i, v_i, entity_emb, relation_emb):
    batch, n_hop, n_mem = h_i.shape
    dim = entity_emb.shape[1]
    h_flat = h_i.reshape(B // CHUNK, CHUNK)
    r_flat = R_i.reshape(B // CHUNK, CHUNK)
    t_flat = t_i.reshape(B // CHUNK, CHUNK)
    rel_flat = relation_emb.reshape(-1, RDIM)
    hs, rs, ts, vs = _run(h_flat, r_flat, t_flat, v_i, entity_emb, rel_flat)
    return (hs.reshape(batch, n_hop, n_mem, dim),
            rs.reshape(batch, n_hop, n_mem, dim, dim),
            ts.reshape(batch, n_hop, n_mem, dim),
            vs)
